# baseline (speedup 1.0000x reference)
import functools

import jax
import jax.numpy as jnp
from jax import lax
from jax.experimental import pallas as pl
from jax.experimental.pallas import tpu as pltpu
from jax.experimental.pallas import tpu_sc as plsc

_B, _N, _D = 16, 4096, 3
_G, _K = 256, 32
_BIG = 1e30
_NSUB = 32
_RPW = _B * _G // _NSUB


def _fps_body(x_ref, y_ref, z_ref, cidx_ref, cx_ref, cy_ref, cz_ref, dist_ref):
    x = x_ref[0]
    y = y_ref[0]
    z = z_ref[0]
    iota_n = jax.lax.broadcasted_iota(jnp.int32, (_B, _N), 1)
    iota_g = jax.lax.broadcasted_iota(jnp.int32, (_B, _G), 1)
    dist_ref[...] = jnp.full((_B, _N), 1e10, jnp.float32)
    cidx_ref[...] = jnp.zeros((_B, _G), jnp.int32)
    cx_ref[...] = jnp.zeros((_B, _G), jnp.float32)
    cy_ref[...] = jnp.zeros((_B, _G), jnp.float32)
    cz_ref[...] = jnp.zeros((_B, _G), jnp.float32)

    def body(i, carry):
        dist = dist_ref[...]
        m = jnp.max(dist, axis=1, keepdims=True)
        far = jnp.min(jnp.where(dist == m, iota_n, _N), axis=1, keepdims=True)
        oh_i = (iota_g == i).astype(jnp.int32)
        oh_f = oh_i.astype(jnp.float32)
        cidx_ref[...] = cidx_ref[...] + oh_i * far
        sel = iota_n == far
        fx = jnp.sum(jnp.where(sel, x, 0.0), axis=1, keepdims=True)
        fy = jnp.sum(jnp.where(sel, y, 0.0), axis=1, keepdims=True)
        fz = jnp.sum(jnp.where(sel, z, 0.0), axis=1, keepdims=True)
        cx_ref[...] = cx_ref[...] + oh_f * fx
        cy_ref[...] = cy_ref[...] + oh_f * fy
        cz_ref[...] = cz_ref[...] + oh_f * fz
        dx = x - fx
        dy = y - fy
        dz = z - fz
        d = (dx * dx + dy * dy) + dz * dz
        dist_ref[...] = jnp.minimum(dist, d)
        return carry

    jax.lax.fori_loop(0, _G, body, 0)


def _dist_body(x_ref, y_ref, z_ref, cx_ref, cy_ref, cz_ref, c3_ref, p3t_ref,
               d2_ref):
    x = x_ref[0]
    y = y_ref[0]
    z = z_ref[0]
    cx = cx_ref[0]
    cy = cy_ref[0]
    cz = cz_ref[0]
    psq = (x * x + y * y) + z * z
    csq = (cx * cx + cy * cy) + cz * cz
    dot = jax.lax.dot_general(
        c3_ref[0], p3t_ref[0], (((1,), (0,)), ((), ())),
        precision=jax.lax.Precision.DEFAULT,
        preferred_element_type=jnp.float32)
    d2_ref[0] = (csq + psq) - 2.0 * dot


def _ffs(mask):
    v = plsc.all_reduce_ffs(mask)
    return jnp.min(v) if v.ndim else v


def _sc_topk_body(d2_hbm, idx_hbm, d2_v, cm_v, cm2_v, idx_v):
    wid = lax.axis_index("s") * 2 + lax.axis_index("c")
    base_row = wid * _RPW
    iota16 = jax.lax.broadcasted_iota(jnp.int32, (16,), 0)

    def row_body(r, carry):
        row = base_row + r
        off = pl.multiple_of(row * _N, _N)
        pltpu.sync_copy(d2_hbm.at[pl.ds(off, _N)], d2_v)

        def slot_body(j, c2):
            idx0 = j * 256 + iota16 * 16

            def inner(t, acc):
                row = plsc.load_gather(d2_v, [idx0 + t])
                return jnp.minimum(acc, row)

            acc = jax.lax.fori_loop(0, 16, inner,
                                    jnp.full((16,), _BIG, jnp.float32))
            cm_v[pl.ds(j * 16, 16)] = acc
            cm2_vec = cm2_v[...]
            cm2_v[...] = jnp.where(iota16 == j, jnp.full((16,), jnp.min(acc)),
                                   cm2_vec)
            return c2

        jax.lax.fori_loop(0, 16, slot_body, 0)

        def k_body(k, carry2):
            c2 = cm2_v[...]
            m = jnp.min(c2)
            jstar = _ffs(c2 == m)
            cmv = cm_v[pl.ds(jstar * 16, 16)]
            cpos = _ffs(cmv == m)
            cstar = jstar * 16 + cpos
            ch = d2_v[pl.ds(cstar * 16, 16)]
            lane = _ffs(ch == m)
            gidx = cstar * 16 + lane
            slot = r * _K + (k // 16) * 16
            cur = idx_v[pl.ds(slot, 16)]
            idx_v[pl.ds(slot, 16)] = jnp.where(
                iota16 == k % 16, jnp.full((16,), gidx, jnp.int32), cur)
            ch2 = jnp.where(iota16 == lane, jnp.full((16,), _BIG), ch)
            d2_v[pl.ds(cstar * 16, 16)] = ch2
            cmv2 = jnp.where(iota16 == cpos, jnp.full((16,), jnp.min(ch2)),
                             cmv)
            cm_v[pl.ds(jstar * 16, 16)] = cmv2
            cm2b = cm2_v[...]
            cm2_v[...] = jnp.where(iota16 == jstar,
                                   jnp.full((16,), jnp.min(cmv2)), cm2b)
            return carry2

        jax.lax.fori_loop(0, _K, k_body, 0)
        return carry

    jax.lax.fori_loop(0, _RPW, row_body, 0)
    out_off = pl.multiple_of(base_row * _K, _RPW * _K)
    pltpu.sync_copy(idx_v, idx_hbm.at[pl.ds(out_off, _RPW * _K)])


def _make_sc_topk():
    return pl.kernel(
        _sc_topk_body,
        out_type=jax.ShapeDtypeStruct((_B * _G * _K,), jnp.int32),
        mesh=plsc.VectorSubcoreMesh(core_axis_name="c", subcore_axis_name="s"),
        compiler_params=pltpu.CompilerParams(needs_layout_passes=False),
        scratch_types=[
            pltpu.VMEM((_N,), jnp.float32),
            pltpu.VMEM((_G,), jnp.float32),
            pltpu.VMEM((16,), jnp.float32),
            pltpu.VMEM((_RPW * _K,), jnp.int32),
        ],
    )


def kernel(xyz):
    xt = jnp.transpose(xyz, (2, 0, 1))
    x3 = xt[:, None]
    x = x3[0]
    y = x3[1]
    z = x3[2]

    fps = pl.pallas_call(
        _fps_body,
        grid=(1,),
        in_specs=[pl.BlockSpec((1, _B, _N), lambda i: (0, 0, 0))] * 3,
        out_specs=[pl.BlockSpec((_B, _G), lambda i: (0, 0))] * 4,
        out_shape=[
            jax.ShapeDtypeStruct((_B, _G), jnp.int32),
            jax.ShapeDtypeStruct((_B, _G), jnp.float32),
            jax.ShapeDtypeStruct((_B, _G), jnp.float32),
            jax.ShapeDtypeStruct((_B, _G), jnp.float32),
        ],
        scratch_shapes=[pltpu.VMEM((_B, _N), jnp.float32)],
    )
    cidx, cx, cy, cz = fps(x, y, z)

    dist = pl.pallas_call(
        _dist_body,
        grid=(_B,),
        in_specs=[
            pl.BlockSpec((1, 1, _N), lambda i: (i, 0, 0)),
            pl.BlockSpec((1, 1, _N), lambda i: (i, 0, 0)),
            pl.BlockSpec((1, 1, _N), lambda i: (i, 0, 0)),
            pl.BlockSpec((1, _G, 1), lambda i: (i, 0, 0)),
            pl.BlockSpec((1, _G, 1), lambda i: (i, 0, 0)),
            pl.BlockSpec((1, _G, 1), lambda i: (i, 0, 0)),
            pl.BlockSpec((1, _G, _D), lambda i: (i, 0, 0)),
            pl.BlockSpec((1, _D, _N), lambda i: (i, 0, 0)),
        ],
        out_specs=pl.BlockSpec((1, _G, _N), lambda i: (i, 0, 0)),
        out_shape=jax.ShapeDtypeStruct((_B, _G, _N), jnp.float32),
    )
    center = jnp.stack([cx, cy, cz], axis=-1)
    p3t = jnp.transpose(xyz, (0, 2, 1))
    d2 = dist(
        x.reshape(_B, 1, _N), y.reshape(_B, 1, _N), z.reshape(_B, 1, _N),
        cx[:, :, None], cy[:, :, None], cz[:, :, None],
        center, p3t,
    )
    idx = _make_sc_topk()(d2.reshape(_B * _G * _N)).reshape(_B, _G, _K)
    return (idx, cidx, center)


# speedup vs baseline: 1.1098x; 1.1098x over previous
import functools

import jax
import jax.numpy as jnp
from jax import lax
from jax.experimental import pallas as pl
from jax.experimental.pallas import tpu as pltpu
from jax.experimental.pallas import tpu_sc as plsc

_B, _N, _D = 16, 4096, 3
_G, _K = 256, 32
_BIG = 1e30
_NSUB = 32
_RPW = _B * _G // _NSUB


def _fps_body(x_ref, y_ref, z_ref, cidx_ref, cx_ref, cy_ref, cz_ref, dist_ref):
    x = x_ref[0]
    y = y_ref[0]
    z = z_ref[0]
    iota_n = jax.lax.broadcasted_iota(jnp.int32, (_B, _N), 1)
    iota_g = jax.lax.broadcasted_iota(jnp.int32, (_B, _G), 1)
    dist_ref[...] = jnp.full((_B, _N), 1e10, jnp.float32)
    cidx_ref[...] = jnp.zeros((_B, _G), jnp.int32)
    cx_ref[...] = jnp.zeros((_B, _G), jnp.float32)
    cy_ref[...] = jnp.zeros((_B, _G), jnp.float32)
    cz_ref[...] = jnp.zeros((_B, _G), jnp.float32)

    def body(i, carry):
        dist = dist_ref[...]
        m = jnp.max(dist, axis=1, keepdims=True)
        far = jnp.min(jnp.where(dist == m, iota_n, _N), axis=1, keepdims=True)
        oh_i = (iota_g == i).astype(jnp.int32)
        oh_f = oh_i.astype(jnp.float32)
        cidx_ref[...] = cidx_ref[...] + oh_i * far
        sel = iota_n == far
        fx = jnp.sum(jnp.where(sel, x, 0.0), axis=1, keepdims=True)
        fy = jnp.sum(jnp.where(sel, y, 0.0), axis=1, keepdims=True)
        fz = jnp.sum(jnp.where(sel, z, 0.0), axis=1, keepdims=True)
        cx_ref[...] = cx_ref[...] + oh_f * fx
        cy_ref[...] = cy_ref[...] + oh_f * fy
        cz_ref[...] = cz_ref[...] + oh_f * fz
        dx = x - fx
        dy = y - fy
        dz = z - fz
        d = (dx * dx + dy * dy) + dz * dz
        dist_ref[...] = jnp.minimum(dist, d)
        return carry

    jax.lax.fori_loop(0, _G, body, 0)


def _dist_body(x_ref, y_ref, z_ref, cx_ref, cy_ref, cz_ref, c3_ref, p3t_ref,
               d2_ref):
    x = x_ref[0]
    y = y_ref[0]
    z = z_ref[0]
    cx = cx_ref[0]
    cy = cy_ref[0]
    cz = cz_ref[0]
    psq = (x * x + y * y) + z * z
    csq = (cx * cx + cy * cy) + cz * cz
    dot = jax.lax.dot_general(
        c3_ref[0], p3t_ref[0], (((1,), (0,)), ((), ())),
        precision=jax.lax.Precision.DEFAULT,
        preferred_element_type=jnp.float32)
    d2_ref[0] = (csq + psq) - 2.0 * dot


def _ffs(mask):
    v = plsc.all_reduce_ffs(mask)
    return jnp.min(v) if v.ndim else v


def _sc_topk_body(d2_hbm, idx_hbm, d2_v, cm_v, cm2_v, idx_v):
    wid = lax.axis_index("s") * 2 + lax.axis_index("c")
    base_row = wid * _RPW
    iota16 = jax.lax.broadcasted_iota(jnp.int32, (16,), 0)

    def row_body(r, carry):
        row = base_row + r
        off = pl.multiple_of(row * _N, _N)
        pltpu.sync_copy(d2_hbm.at[pl.ds(off, _N)], d2_v)

        def slot_body(j, c2):
            def inner(t, acc):
                ch = d2_v[pl.ds((j * 16 + t) * 16, 16)]
                mv = jnp.min(ch)
                return jnp.where(iota16 == t, jnp.full((16,), mv), acc)

            acc = jax.lax.fori_loop(0, 16, inner,
                                    jnp.full((16,), _BIG, jnp.float32))
            cm_v[pl.ds(j * 16, 16)] = acc
            cm2_vec = cm2_v[...]
            cm2_v[...] = jnp.where(iota16 == j, jnp.full((16,), jnp.min(acc)),
                                   cm2_vec)
            return c2

        jax.lax.fori_loop(0, 16, slot_body, 0)

        def k_body(k, carry2):
            c2 = cm2_v[...]
            m = jnp.min(c2)
            jstar = _ffs(c2 == m)
            cmv = cm_v[pl.ds(jstar * 16, 16)]
            cpos = _ffs(cmv == m)
            cstar = jstar * 16 + cpos
            ch = d2_v[pl.ds(cstar * 16, 16)]
            lane = _ffs(ch == m)
            gidx = cstar * 16 + lane
            slot = r * _K + (k // 16) * 16
            cur = idx_v[pl.ds(slot, 16)]
            idx_v[pl.ds(slot, 16)] = jnp.where(
                iota16 == k % 16, jnp.full((16,), gidx, jnp.int32), cur)
            ch2 = jnp.where(iota16 == lane, jnp.full((16,), _BIG), ch)
            d2_v[pl.ds(cstar * 16, 16)] = ch2
            cmv2 = jnp.where(iota16 == cpos, jnp.full((16,), jnp.min(ch2)),
                             cmv)
            cm_v[pl.ds(jstar * 16, 16)] = cmv2
            cm2b = cm2_v[...]
            cm2_v[...] = jnp.where(iota16 == jstar,
                                   jnp.full((16,), jnp.min(cmv2)), cm2b)
            return carry2

        jax.lax.fori_loop(0, _K, k_body, 0)
        return carry

    jax.lax.fori_loop(0, _RPW, row_body, 0)
    out_off = pl.multiple_of(base_row * _K, _RPW * _K)
    pltpu.sync_copy(idx_v, idx_hbm.at[pl.ds(out_off, _RPW * _K)])


def _make_sc_topk():
    return pl.kernel(
        _sc_topk_body,
        out_type=jax.ShapeDtypeStruct((_B * _G * _K,), jnp.int32),
        mesh=plsc.VectorSubcoreMesh(core_axis_name="c", subcore_axis_name="s"),
        compiler_params=pltpu.CompilerParams(needs_layout_passes=False),
        scratch_types=[
            pltpu.VMEM((_N,), jnp.float32),
            pltpu.VMEM((_G,), jnp.float32),
            pltpu.VMEM((16,), jnp.float32),
            pltpu.VMEM((_RPW * _K,), jnp.int32),
        ],
    )


def kernel(xyz):
    xt = jnp.transpose(xyz, (2, 0, 1))
    x3 = xt[:, None]
    x = x3[0]
    y = x3[1]
    z = x3[2]

    fps = pl.pallas_call(
        _fps_body,
        grid=(1,),
        in_specs=[pl.BlockSpec((1, _B, _N), lambda i: (0, 0, 0))] * 3,
        out_specs=[pl.BlockSpec((_B, _G), lambda i: (0, 0))] * 4,
        out_shape=[
            jax.ShapeDtypeStruct((_B, _G), jnp.int32),
            jax.ShapeDtypeStruct((_B, _G), jnp.float32),
            jax.ShapeDtypeStruct((_B, _G), jnp.float32),
            jax.ShapeDtypeStruct((_B, _G), jnp.float32),
        ],
        scratch_shapes=[pltpu.VMEM((_B, _N), jnp.float32)],
    )
    cidx, cx, cy, cz = fps(x, y, z)

    dist = pl.pallas_call(
        _dist_body,
        grid=(_B,),
        in_specs=[
            pl.BlockSpec((1, 1, _N), lambda i: (i, 0, 0)),
            pl.BlockSpec((1, 1, _N), lambda i: (i, 0, 0)),
            pl.BlockSpec((1, 1, _N), lambda i: (i, 0, 0)),
            pl.BlockSpec((1, _G, 1), lambda i: (i, 0, 0)),
            pl.BlockSpec((1, _G, 1), lambda i: (i, 0, 0)),
            pl.BlockSpec((1, _G, 1), lambda i: (i, 0, 0)),
            pl.BlockSpec((1, _G, _D), lambda i: (i, 0, 0)),
            pl.BlockSpec((1, _D, _N), lambda i: (i, 0, 0)),
        ],
        out_specs=pl.BlockSpec((1, _G, _N), lambda i: (i, 0, 0)),
        out_shape=jax.ShapeDtypeStruct((_B, _G, _N), jnp.float32),
    )
    center = jnp.stack([cx, cy, cz], axis=-1)
    p3t = jnp.transpose(xyz, (0, 2, 1))
    d2 = dist(
        x.reshape(_B, 1, _N), y.reshape(_B, 1, _N), z.reshape(_B, 1, _N),
        cx[:, :, None], cy[:, :, None], cz[:, :, None],
        center, p3t,
    )
    idx = _make_sc_topk()(d2.reshape(_B * _G * _N)).reshape(_B, _G, _K)
    return (idx, cidx, center)
